# Initial kernel scaffold; baseline (speedup 1.0000x reference)
import jax
import jax.numpy as jnp
from jax.experimental import pallas as pl


def kernel(x, edge_index_l0, edge_index_l1, pos_edge_index, neg_edge_index, W_in, b_in, lora_A, lora_B, pn_g, pn_b, Ws0, Wn0, bb0, ln0_g, ln0_b, Ws1, Wn1, bb1, ln1_g, ln1_b):
    raise NotImplementedError("write your pallas kernel here")


# speedup vs baseline: 2.4644x; 2.4644x over previous
import functools

import jax
import jax.numpy as jnp
from jax import lax
from jax.experimental import pallas as pl
from jax.experimental.pallas import tpu as pltpu
from jax.experimental.pallas import tpu_sc as plsc

N = 10000
D = 128
H = 128
E = 320000
EP = 10000

NC = 2
NS = 16
NW = NC * NS
NPAD = 10240
RPS = NPAD // NS
CH = 128
EPW = 10240
E_PAD = NW * EPW
NCH_E = EPW // CH
EPP = 10240
GTOT = 4 * EPP
GPW = GTOT // NW
NCH_G = GPW // CH


def _sc_mesh():
    return plsc.VectorSubcoreMesh(core_axis_name="c", subcore_axis_name="s",
                                  num_cores=NC, num_subcores=NS)


@functools.partial(
    pl.kernel,
    out_type=(jax.ShapeDtypeStruct((NC * NPAD, H), jnp.float32),
              jax.ShapeDtypeStruct((NC * NPAD, H), jnp.float32)),
    mesh=_sc_mesh(),
    scratch_types=[
        pltpu.VMEM((CH,), jnp.int32),
        pltpu.VMEM((CH,), jnp.int32),
        pltpu.VMEM((CH, H), jnp.float32),
        pltpu.SemaphoreType.DMA,
        pltpu.VMEM_SHARED((NPAD, H), jnp.float32),
    ],
)
def _sc_aggregate(h_hbm, src_hbm, dst_hbm, zrow_hbm, ones_hbm,
                  agg_out, deg_out,
                  sidx_v, didx_v, rows_v,
                  sem, agg_sh):
    cid = lax.axis_index("c")
    sid = lax.axis_index("s")
    wid = sid * NC + cid
    r0 = sid * RPS
    pltpu.sync_copy(zrow_hbm, rows_v)

    def zbody(j, carry):
        pltpu.sync_copy(rows_v, agg_sh.at[pl.ds(r0 + j * CH, CH)])
        return carry

    lax.fori_loop(0, RPS // CH, zbody, 0)
    plsc.subcore_barrier()

    ebase = wid * EPW

    def body(i, carry):
        base = ebase + i * CH
        pltpu.sync_copy(src_hbm.at[pl.ds(base, CH)], sidx_v)
        pltpu.sync_copy(dst_hbm.at[pl.ds(base, CH)], didx_v)
        pltpu.async_copy(h_hbm.at[sidx_v], rows_v, sem).wait()
        pltpu.sync_copy(rows_v, agg_sh.at[didx_v], add=True)
        return carry

    lax.fori_loop(0, NCH_E, body, 0)
    plsc.subcore_barrier()
    o0 = cid * NPAD + r0

    def obody(j, carry):
        pltpu.sync_copy(agg_sh.at[pl.ds(r0 + j * CH, CH)], rows_v)
        pltpu.sync_copy(rows_v, agg_out.at[pl.ds(o0 + j * CH, CH)])
        return carry

    lax.fori_loop(0, RPS // CH, obody, 0)
    pltpu.sync_copy(zrow_hbm, rows_v)

    def zbody2(j, carry):
        pltpu.sync_copy(rows_v, agg_sh.at[pl.ds(r0 + j * CH, CH)])
        return carry

    lax.fori_loop(0, RPS // CH, zbody2, 0)
    plsc.subcore_barrier()
    pltpu.sync_copy(ones_hbm, rows_v)

    def dbody(i, carry):
        base = ebase + i * CH
        pltpu.sync_copy(dst_hbm.at[pl.ds(base, CH)], didx_v)
        pltpu.sync_copy(rows_v, agg_sh.at[didx_v], add=True)
        return carry

    lax.fori_loop(0, NCH_E, dbody, 0)
    plsc.subcore_barrier()

    def dout(j, carry):
        pltpu.sync_copy(agg_sh.at[pl.ds(r0 + j * CH, CH)], rows_v)
        pltpu.sync_copy(rows_v, deg_out.at[pl.ds(o0 + j * CH, CH)])
        return carry

    lax.fori_loop(0, RPS // CH, dout, 0)


@functools.partial(
    pl.kernel,
    out_type=jax.ShapeDtypeStruct((GTOT, H), jnp.float32),
    mesh=_sc_mesh(),
    scratch_types=[
        pltpu.VMEM((CH,), jnp.int32),
        pltpu.VMEM((CH, H), jnp.float32),
        pltpu.SemaphoreType.DMA,
    ],
)
def _sc_gather_rows(h_hbm, idx_hbm, g_out, idx_v, rows_v, sem):
    cid = lax.axis_index("c")
    sid = lax.axis_index("s")
    wid = sid * NC + cid
    base0 = wid * GPW

    def body(i, carry):
        b = base0 + i * CH
        pltpu.sync_copy(idx_hbm.at[pl.ds(b, CH)], idx_v)
        pltpu.async_copy(h_hbm.at[idx_v], rows_v, sem).wait()
        pltpu.sync_copy(rows_v, g_out.at[pl.ds(b, CH)])
        return carry

    lax.fori_loop(0, NCH_G, body, 0)


def _ln(t, g, b):
    m = jnp.mean(t, axis=-1, keepdims=True)
    v = jnp.mean((t - m) ** 2, axis=-1, keepdims=True)
    return (t - m) * lax.rsqrt(v + 1e-5) * g + b


def _inproj_body(x_ref, w_ref, b_ref, la_ref, lb_ref, g_ref, be_ref, o_ref):
    xb = x_ref[...]
    t = jnp.dot(xb, w_ref[...], preferred_element_type=jnp.float32)
    t = t + jnp.dot(jnp.dot(xb, la_ref[...], preferred_element_type=jnp.float32),
                    lb_ref[...], preferred_element_type=jnp.float32)
    t = t + b_ref[...]
    o_ref[...] = _ln(t, g_ref[...], be_ref[...])


_BLK_IN = 512


def _inproj(xpad, W_in, b_in, lora_A, lora_B, pn_g, pn_b):
    full = lambda s: pl.BlockSpec(s, lambda i: (0, 0))
    return pl.pallas_call(
        _inproj_body,
        grid=(NPAD // _BLK_IN,),
        in_specs=[
            pl.BlockSpec((_BLK_IN, D), lambda i: (i, 0)),
            full((D, H)), full((1, H)), full((D, 16)), full((16, H)),
            full((1, H)), full((1, H)),
        ],
        out_specs=pl.BlockSpec((_BLK_IN, H), lambda i: (i, 0)),
        out_shape=jax.ShapeDtypeStruct((NPAD, H), jnp.float32),
    )(xpad, W_in, b_in, lora_A, lora_B, pn_g, pn_b)


def _layer_body(resid, h_ref, a_ref, d_ref, ws_ref, wn_ref, bb_ref,
                g_ref, be_ref, o_ref):
    hb = h_ref[...]
    agg = a_ref[0] + a_ref[1]
    deg = d_ref[0, :, 0:1] + d_ref[1, :, 0:1]
    agg = agg / jnp.maximum(deg, 1.0)
    t = (jnp.dot(hb, ws_ref[...], preferred_element_type=jnp.float32)
         + jnp.dot(agg, wn_ref[...], preferred_element_type=jnp.float32)
         + bb_ref[...])
    t = jnp.maximum(t, 0.0)
    if resid:
        t = t + hb
    o_ref[...] = _ln(t, g_ref[...], be_ref[...])


_BLK_L = 640


def _layer(resid, h, agg2, deg2, Ws, Wn, bb, g, be):
    full = lambda s: pl.BlockSpec(s, lambda i: (0, 0))
    return pl.pallas_call(
        functools.partial(_layer_body, resid),
        grid=(NPAD // _BLK_L,),
        in_specs=[
            pl.BlockSpec((_BLK_L, H), lambda i: (i, 0)),
            pl.BlockSpec((NC, _BLK_L, H), lambda i: (0, i, 0)),
            pl.BlockSpec((NC, _BLK_L, H), lambda i: (0, i, 0)),
            full((H, H)), full((H, H)), full((1, H)), full((1, H)), full((1, H)),
        ],
        out_specs=pl.BlockSpec((_BLK_L, H), lambda i: (i, 0)),
        out_shape=jax.ShapeDtypeStruct((NPAD, H), jnp.float32),
    )(h, agg2, deg2, Ws, Wn, bb, g, be)


def _score_body(g_ref, p_ref, n_ref):
    p_ref[...] = jnp.sum(g_ref[0] * g_ref[1], axis=-1)
    n_ref[...] = jnp.sum(g_ref[2] * g_ref[3], axis=-1)


_BLK_S = 512


def _score(g4):
    nblk = EPP // _BLK_S
    return pl.pallas_call(
        _score_body,
        grid=(nblk,),
        in_specs=[pl.BlockSpec((4, _BLK_S, H), lambda i: (0, i, 0))],
        out_specs=(pl.BlockSpec((_BLK_S,), lambda i: (i,)),
                   pl.BlockSpec((_BLK_S,), lambda i: (i,))),
        out_shape=(jax.ShapeDtypeStruct((EPP,), jnp.float32),
                   jax.ShapeDtypeStruct((EPP,), jnp.float32)),
    )(g4)


def kernel(x, edge_index_l0, edge_index_l1, pos_edge_index, neg_edge_index,
           W_in, b_in, lora_A, lora_B, pn_g, pn_b,
           Ws0, Wn0, bb0, ln0_g, ln0_b,
           Ws1, Wn1, bb1, ln1_g, ln1_b):
    f32 = jnp.float32
    xpad = jnp.pad(x, ((0, NPAD - N), (0, 0)))
    h = _inproj(xpad, W_in, b_in.reshape(1, H), lora_A, lora_B,
                pn_g.reshape(1, H), pn_b.reshape(1, H))

    zrow = jnp.zeros((CH, H), f32)
    ones128 = jnp.ones((CH, H), f32)
    pad_e = E_PAD - E
    src_pad = jnp.zeros((pad_e,), jnp.int32)
    dst_pad = jnp.full((pad_e,), N, jnp.int32)

    layers = ((edge_index_l0, Ws0, Wn0, bb0, ln0_g, ln0_b, False),
              (edge_index_l1, Ws1, Wn1, bb1, ln1_g, ln1_b, True))
    for ei, Ws, Wn, bb, g, be, resid in layers:
        src = jnp.concatenate([ei[0], src_pad])
        dst = jnp.concatenate([ei[1], dst_pad])
        agg2, deg2 = _sc_aggregate(h, src, dst, zrow, ones128)
        h = _layer(resid, h, agg2.reshape(NC, NPAD, H),
                   deg2.reshape(NC, NPAD, H),
                   Ws, Wn, bb.reshape(1, H), g.reshape(1, H), be.reshape(1, H))

    zp = jnp.zeros((EPP - EP,), jnp.int32)
    idx_all = jnp.concatenate([pos_edge_index[0], zp, pos_edge_index[1], zp,
                               neg_edge_index[0], zp, neg_edge_index[1], zp])
    gath = _sc_gather_rows(h, idx_all)
    pos2, neg2 = _score(gath.reshape(4, EPP, H))
    return (pos2[:EP], neg2[:EP])


# speedup vs baseline: 2.8991x; 1.1764x over previous
import functools

import jax
import jax.numpy as jnp
from jax import lax
from jax.experimental import pallas as pl
from jax.experimental.pallas import tpu as pltpu
from jax.experimental.pallas import tpu_sc as plsc

N = 10000
D = 128
H = 128
E = 320000
EP = 10000

NC = 2
NS = 16
NW = NC * NS
NPAD = 10240
RPS = NPAD // NS
CH = 128
EPW = 10240
E_PAD = NW * EPW
NCH_E = EPW // CH
IB = 8
EPP = 10240
GTOT = 4 * EPP
GPW = GTOT // NW
NCH_G = GPW // CH


def _sc_mesh():
    return plsc.VectorSubcoreMesh(core_axis_name="c", subcore_axis_name="s",
                                  num_cores=NC, num_subcores=NS)


@functools.partial(
    pl.kernel,
    out_type=(jax.ShapeDtypeStruct((NC * NPAD, H), jnp.float32),
              jax.ShapeDtypeStruct((NC * NPAD, H), jnp.float32)),
    mesh=_sc_mesh(),
    scratch_types=[
        pltpu.VMEM((IB, CH), jnp.int32),
        pltpu.VMEM((IB, CH), jnp.int32),
        pltpu.VMEM((CH, H), jnp.float32),
        pltpu.VMEM((CH, H), jnp.float32),
        pltpu.SemaphoreType.DMA,
        pltpu.SemaphoreType.DMA,
        pltpu.VMEM_SHARED((NPAD, H), jnp.float32),
    ],
)
def _sc_aggregate(h_hbm, src_hbm, dst_hbm, zrow_hbm, ones_hbm,
                  agg_out, deg_out,
                  sidx_b, didx_b, rows_v, rows2_v,
                  sem, sem2, agg_sh):
    cid = lax.axis_index("c")
    sid = lax.axis_index("s")
    wid = sid * NC + cid
    r0 = sid * RPS
    pltpu.sync_copy(zrow_hbm, rows_v)

    def zbody(j, carry):
        pltpu.sync_copy(rows_v, agg_sh.at[pl.ds(r0 + j * CH, CH)])
        return carry

    lax.fori_loop(0, RPS // CH, zbody, 0)
    plsc.subcore_barrier()

    crow0 = wid * NCH_E
    bufs = (rows_v, rows2_v)

    def body(gi, carry):
        grow = crow0 + gi * IB
        pltpu.sync_copy(src_hbm.at[pl.ds(grow, IB)], sidx_b)
        pltpu.sync_copy(dst_hbm.at[pl.ds(grow, IB)], didx_b)
        g = [None] * IB
        g[0] = pltpu.async_copy(h_hbm.at[sidx_b.at[0]], bufs[0], sem)
        for k in range(IB):
            g[k].wait()
            if k + 1 < IB:
                g[k + 1] = pltpu.async_copy(h_hbm.at[sidx_b.at[k + 1]],
                                            bufs[(k + 1) % 2], sem)
            pltpu.sync_copy(bufs[k % 2], agg_sh.at[didx_b.at[k]], add=True)
        return carry

    lax.fori_loop(0, NCH_E // IB, body, 0)
    plsc.subcore_barrier()
    o0 = cid * NPAD + r0

    def obody(j, carry):
        pltpu.sync_copy(agg_sh.at[pl.ds(r0 + j * CH, CH)], rows_v)
        pltpu.sync_copy(rows_v, agg_out.at[pl.ds(o0 + j * CH, CH)])
        return carry

    lax.fori_loop(0, RPS // CH, obody, 0)
    pltpu.sync_copy(zrow_hbm, rows_v)

    def zbody2(j, carry):
        pltpu.sync_copy(rows_v, agg_sh.at[pl.ds(r0 + j * CH, CH)])
        return carry

    lax.fori_loop(0, RPS // CH, zbody2, 0)
    plsc.subcore_barrier()
    pltpu.sync_copy(ones_hbm, rows_v)

    def dbody(gi, carry):
        grow = crow0 + gi * IB
        pltpu.sync_copy(dst_hbm.at[pl.ds(grow, IB)], didx_b)
        descs = [pltpu.async_copy(rows_v, agg_sh.at[didx_b.at[k]], sem2,
                                  add=True)
                 for k in range(IB)]
        for dd in descs:
            dd.wait()
        return carry

    lax.fori_loop(0, NCH_E // IB, dbody, 0)
    plsc.subcore_barrier()

    def dout(j, carry):
        pltpu.sync_copy(agg_sh.at[pl.ds(r0 + j * CH, CH)], rows_v)
        pltpu.sync_copy(rows_v, deg_out.at[pl.ds(o0 + j * CH, CH)])
        return carry

    lax.fori_loop(0, RPS // CH, dout, 0)


@functools.partial(
    pl.kernel,
    out_type=jax.ShapeDtypeStruct((GTOT, H), jnp.float32),
    mesh=_sc_mesh(),
    scratch_types=[
        pltpu.VMEM((CH,), jnp.int32),
        pltpu.VMEM((CH, H), jnp.float32),
        pltpu.SemaphoreType.DMA,
    ],
)
def _sc_gather_rows(h_hbm, idx_hbm, g_out, idx_v, rows_v, sem):
    cid = lax.axis_index("c")
    sid = lax.axis_index("s")
    wid = sid * NC + cid
    base0 = wid * GPW

    def body(i, carry):
        b = base0 + i * CH
        pltpu.sync_copy(idx_hbm.at[pl.ds(b, CH)], idx_v)
        pltpu.async_copy(h_hbm.at[idx_v], rows_v, sem).wait()
        pltpu.sync_copy(rows_v, g_out.at[pl.ds(b, CH)])
        return carry

    lax.fori_loop(0, NCH_G, body, 0)


def _ln(t, g, b):
    m = jnp.mean(t, axis=-1, keepdims=True)
    v = jnp.mean((t - m) ** 2, axis=-1, keepdims=True)
    return (t - m) * lax.rsqrt(v + 1e-5) * g + b


def _inproj_body(x_ref, w_ref, b_ref, la_ref, lb_ref, g_ref, be_ref, o_ref):
    xb = x_ref[...]
    t = jnp.dot(xb, w_ref[...], preferred_element_type=jnp.float32)
    t = t + jnp.dot(jnp.dot(xb, la_ref[...], preferred_element_type=jnp.float32),
                    lb_ref[...], preferred_element_type=jnp.float32)
    t = t + b_ref[...]
    o_ref[...] = _ln(t, g_ref[...], be_ref[...])


_BLK_IN = 512


def _inproj(xpad, W_in, b_in, lora_A, lora_B, pn_g, pn_b):
    full = lambda s: pl.BlockSpec(s, lambda i: (0, 0))
    return pl.pallas_call(
        _inproj_body,
        grid=(NPAD // _BLK_IN,),
        in_specs=[
            pl.BlockSpec((_BLK_IN, D), lambda i: (i, 0)),
            full((D, H)), full((1, H)), full((D, 16)), full((16, H)),
            full((1, H)), full((1, H)),
        ],
        out_specs=pl.BlockSpec((_BLK_IN, H), lambda i: (i, 0)),
        out_shape=jax.ShapeDtypeStruct((NPAD, H), jnp.float32),
    )(xpad, W_in, b_in, lora_A, lora_B, pn_g, pn_b)


def _layer_body(resid, h_ref, a_ref, d_ref, ws_ref, wn_ref, bb_ref,
                g_ref, be_ref, o_ref):
    hb = h_ref[...]
    agg = a_ref[0] + a_ref[1]
    deg = d_ref[0, :, 0:1] + d_ref[1, :, 0:1]
    agg = agg / jnp.maximum(deg, 1.0)
    t = (jnp.dot(hb, ws_ref[...], preferred_element_type=jnp.float32)
         + jnp.dot(agg, wn_ref[...], preferred_element_type=jnp.float32)
         + bb_ref[...])
    t = jnp.maximum(t, 0.0)
    if resid:
        t = t + hb
    o_ref[...] = _ln(t, g_ref[...], be_ref[...])


_BLK_L = 640


def _layer(resid, h, agg2, deg2, Ws, Wn, bb, g, be):
    full = lambda s: pl.BlockSpec(s, lambda i: (0, 0))
    return pl.pallas_call(
        functools.partial(_layer_body, resid),
        grid=(NPAD // _BLK_L,),
        in_specs=[
            pl.BlockSpec((_BLK_L, H), lambda i: (i, 0)),
            pl.BlockSpec((NC, _BLK_L, H), lambda i: (0, i, 0)),
            pl.BlockSpec((NC, _BLK_L, H), lambda i: (0, i, 0)),
            full((H, H)), full((H, H)), full((1, H)), full((1, H)), full((1, H)),
        ],
        out_specs=pl.BlockSpec((_BLK_L, H), lambda i: (i, 0)),
        out_shape=jax.ShapeDtypeStruct((NPAD, H), jnp.float32),
    )(h, agg2, deg2, Ws, Wn, bb, g, be)


def _score_body(g_ref, p_ref, n_ref):
    p_ref[...] = jnp.sum(g_ref[0] * g_ref[1], axis=-1)
    n_ref[...] = jnp.sum(g_ref[2] * g_ref[3], axis=-1)


_BLK_S = 512


def _score(g4):
    nblk = EPP // _BLK_S
    return pl.pallas_call(
        _score_body,
        grid=(nblk,),
        in_specs=[pl.BlockSpec((4, _BLK_S, H), lambda i: (0, i, 0))],
        out_specs=(pl.BlockSpec((_BLK_S,), lambda i: (i,)),
                   pl.BlockSpec((_BLK_S,), lambda i: (i,))),
        out_shape=(jax.ShapeDtypeStruct((EPP,), jnp.float32),
                   jax.ShapeDtypeStruct((EPP,), jnp.float32)),
    )(g4)


def kernel(x, edge_index_l0, edge_index_l1, pos_edge_index, neg_edge_index,
           W_in, b_in, lora_A, lora_B, pn_g, pn_b,
           Ws0, Wn0, bb0, ln0_g, ln0_b,
           Ws1, Wn1, bb1, ln1_g, ln1_b):
    f32 = jnp.float32
    xpad = jnp.pad(x, ((0, NPAD - N), (0, 0)))
    h = _inproj(xpad, W_in, b_in.reshape(1, H), lora_A, lora_B,
                pn_g.reshape(1, H), pn_b.reshape(1, H))

    zrow = jnp.zeros((CH, H), f32)
    ones128 = jnp.ones((CH, H), f32)
    pad_e = E_PAD - E
    src_pad = jnp.zeros((pad_e,), jnp.int32)
    dst_pad = jnp.full((pad_e,), N, jnp.int32)

    layers = ((edge_index_l0, Ws0, Wn0, bb0, ln0_g, ln0_b, False),
              (edge_index_l1, Ws1, Wn1, bb1, ln1_g, ln1_b, True))
    for ei, Ws, Wn, bb, g, be, resid in layers:
        src = jnp.concatenate([ei[0], src_pad]).reshape(E_PAD // CH, CH)
        dst = jnp.concatenate([ei[1], dst_pad]).reshape(E_PAD // CH, CH)
        agg2, deg2 = _sc_aggregate(h, src, dst, zrow, ones128)
        h = _layer(resid, h, agg2.reshape(NC, NPAD, H),
                   deg2.reshape(NC, NPAD, H),
                   Ws, Wn, bb.reshape(1, H), g.reshape(1, H), be.reshape(1, H))

    zp = jnp.zeros((EPP - EP,), jnp.int32)
    idx_all = jnp.concatenate([pos_edge_index[0], zp, pos_edge_index[1], zp,
                               neg_edge_index[0], zp, neg_edge_index[1], zp])
    gath = _sc_gather_rows(h, idx_all)
    pos2, neg2 = _score(gath.reshape(4, EPP, H))
    return (pos2[:EP], neg2[:EP])
